# baseline (speedup 1.0000x reference)
import jax
import jax.numpy as jnp
from jax.experimental import pallas as pl
from jax.experimental.pallas import tpu as pltpu

N = 512
H = 128
B = 128
NT = N // B

_LOG2E = 1.4426950408889634
_LN2 = 0.6931471805599453


def _adj_kernel(x_ref, w1_ref, b1_ref, w2_ref, b2_ref, out_ref, u_s, v_s):
    ti = pl.program_id(0)
    tj = pl.program_id(1)

    @pl.when(jnp.logical_and(ti == 0, tj == 0))
    def _init():
        x = x_ref[...]
        w1 = w1_ref[...]
        dn = (((1,), (1,)), ((), ()))
        u = jax.lax.dot_general(x, w1[:, :H], dn,
                                preferred_element_type=jnp.float32)
        v = jax.lax.dot_general(x, w1[:, H:], dn,
                                preferred_element_type=jnp.float32)
        u_s[...] = (u + b1_ref[...]) * _LOG2E
        v_s[...] = v * _LOG2E

    w2col = w2_ref[...].reshape(H, 1)
    b2 = b2_ref[0, 0] - jnp.sum(w2col)

    def scores(pre):
        act = jnp.where(pre > 0, pre * _LN2 + 1.0, jnp.exp2(pre))
        s = jax.lax.dot_general(act.reshape(B * B, H), w2col,
                                (((1,), (0,)), ((), ())),
                                preferred_element_type=jnp.float32)
        return s.reshape(B, B) + b2

    @pl.when(ti < tj)
    def _upper():
        ub = u_s[pl.ds(ti * B, B), :]
        vj = v_s[pl.ds(tj * B, B), :]
        s = scores(ub[:, None, :] + vj[None, :, :])
        out_ref[pl.ds(ti * B, B), pl.ds(tj * B, B)] = s
        out_ref[pl.ds(tj * B, B), pl.ds(ti * B, B)] = s.T

    @pl.when(jnp.logical_and(ti == tj, ti % 2 == 0))
    def _diag_pair():
        a = ti
        b = ti + 1
        ua = u_s[pl.ds(a * B, B), :]
        va = v_s[pl.ds(a * B, B), :]
        ub = u_s[pl.ds(b * B, B), :]
        vb = v_s[pl.ds(b * B, B), :]
        r = jax.lax.broadcasted_iota(jnp.int32, (B, B), 0)
        c = jax.lax.broadcasted_iota(jnp.int32, (B, B), 1)
        r3 = jax.lax.broadcasted_iota(jnp.int32, (B, B, H), 0)
        c3 = jax.lax.broadcasted_iota(jnp.int32, (B, B, H), 1)
        pre = jnp.where(r3 < c3, ua[:, None, :] + va[None, :, :],
                        vb[:, None, :] + ub[None, :, :])
        s = scores(pre)
        su = jnp.where(r < c, s, 0.0)
        sl = jnp.where(r > c, s, 0.0)
        out_ref[pl.ds(a * B, B), pl.ds(a * B, B)] = su + su.T
        out_ref[pl.ds(b * B, B), pl.ds(b * B, B)] = sl + sl.T


def kernel(x, W1, b1, W2, b2):
    b1r = b1.reshape(1, H)
    b2r = b2.reshape(1, 1)
    return pl.pallas_call(
        _adj_kernel,
        grid=(NT, NT),
        in_specs=[
            pl.BlockSpec((N, H), lambda i, j: (0, 0)),
            pl.BlockSpec((H, 2 * H), lambda i, j: (0, 0)),
            pl.BlockSpec((1, H), lambda i, j: (0, 0)),
            pl.BlockSpec((1, H), lambda i, j: (0, 0)),
            pl.BlockSpec((1, 1), lambda i, j: (0, 0)),
        ],
        out_specs=pl.BlockSpec((N, N), lambda i, j: (0, 0)),
        out_shape=jax.ShapeDtypeStruct((N, N), jnp.float32),
        scratch_shapes=[pltpu.VMEM((N, H), jnp.float32),
                        pltpu.VMEM((N, H), jnp.float32)],
    )(x, W1, b1r, W2, b2r)


# speedup vs baseline: 1.0843x; 1.0843x over previous
import jax
import jax.numpy as jnp
from jax.experimental import pallas as pl
from jax.experimental.pallas import tpu as pltpu

N = 512
H = 128
B = 128
NT = N // B


def _elu(z):
    return jnp.where(z > 0, z, jnp.exp(z) - 1.0)


def _adj_kernel(x_ref, w1_ref, b1_ref, w2_ref, b2_ref, out_ref, u_s, v_s):
    ti = pl.program_id(0)
    tj = pl.program_id(1)

    @pl.when(jnp.logical_and(ti == 0, tj == 0))
    def _init():
        x = x_ref[...]
        w1 = w1_ref[...]
        dn = (((1,), (1,)), ((), ()))
        u = jax.lax.dot_general(x, w1[:, :H], dn,
                                preferred_element_type=jnp.float32)
        v = jax.lax.dot_general(x, w1[:, H:], dn,
                                preferred_element_type=jnp.float32)
        u_s[...] = u + b1_ref[...]
        v_s[...] = v

    w2col = w2_ref[...].reshape(H, 1)
    b2 = b2_ref[0, 0]

    def reduce_w2(act):
        s = jax.lax.dot_general(act.reshape(B * B, H), w2col,
                                (((1,), (0,)), ((), ())),
                                preferred_element_type=jnp.float32)
        return s.reshape(B, B)

    @pl.when(ti < tj)
    def _upper():
        ub = u_s[pl.ds(ti * B, B), :]
        vj = v_s[pl.ds(tj * B, B), :]
        pre = ub[:, None, :] + vj[None, :, :]
        s = reduce_w2(_elu(pre)) + b2
        out_ref[pl.ds(ti * B, B), pl.ds(tj * B, B)] = s
        out_ref[pl.ds(tj * B, B), pl.ds(ti * B, B)] = s.T

    @pl.when(jnp.logical_and(ti == tj, ti % 2 == 0))
    def _diag_pair():
        a = ti
        b = ti + 1
        ua = u_s[pl.ds(a * B, B), :]
        va = v_s[pl.ds(a * B, B), :]
        ub = u_s[pl.ds(b * B, B), :]
        vb = v_s[pl.ds(b * B, B), :]
        r = jax.lax.broadcasted_iota(jnp.int32, (B, B), 0)
        c = jax.lax.broadcasted_iota(jnp.int32, (B, B), 1)
        r3 = jax.lax.broadcasted_iota(jnp.int32, (B, B, H), 0)
        c3 = jax.lax.broadcasted_iota(jnp.int32, (B, B, H), 1)
        pre = jnp.where(r3 < c3, ua[:, None, :] + va[None, :, :],
                        vb[:, None, :] + ub[None, :, :])
        s = reduce_w2(_elu(pre)) + b2
        su = jnp.where(r < c, s, 0.0)
        sl = jnp.where(r > c, s, 0.0)
        out_ref[pl.ds(a * B, B), pl.ds(a * B, B)] = su + su.T
        out_ref[pl.ds(b * B, B), pl.ds(b * B, B)] = sl + sl.T


def kernel(x, W1, b1, W2, b2):
    b1r = b1.reshape(1, H)
    b2r = b2.reshape(1, 1)
    return pl.pallas_call(
        _adj_kernel,
        grid=(NT, NT),
        in_specs=[
            pl.BlockSpec((N, H), lambda i, j: (0, 0)),
            pl.BlockSpec((H, 2 * H), lambda i, j: (0, 0)),
            pl.BlockSpec((1, H), lambda i, j: (0, 0)),
            pl.BlockSpec((1, H), lambda i, j: (0, 0)),
            pl.BlockSpec((1, 1), lambda i, j: (0, 0)),
        ],
        out_specs=pl.BlockSpec((N, N), lambda i, j: (0, 0)),
        out_shape=jax.ShapeDtypeStruct((N, N), jnp.float32),
        scratch_shapes=[pltpu.VMEM((N, H), jnp.float32),
                        pltpu.VMEM((N, H), jnp.float32)],
    )(x, W1, b1r, W2, b2r)
